# baseline (speedup 1.0000x reference)
import functools

import jax
import jax.numpy as jnp
from jax import lax
from jax.experimental import pallas as pl
from jax.experimental.pallas import tpu as pltpu
from jax.experimental.pallas import tpu_sc as plsc

_info = plsc.get_sparse_core_info()
_NC, _NS, _L = _info.num_cores, _info.num_subcores, _info.num_lanes
_NW = _NC * _NS
_CHUNK = 128
_FT = 128


def _full(v):
    return jnp.full((_L,), v, jnp.int32)


@functools.cache
def _make_sc_kernel(B, V):
    bpw = B // _NW
    nchunks = bpw // _CHUNK
    vm = V // _FT * _FT
    mesh = plsc.VectorSubcoreMesh(core_axis_name="c", subcore_axis_name="s")

    @functools.partial(
        pl.kernel,
        mesh=mesh,
        compiler_params=pltpu.CompilerParams(
            needs_layout_passes=False, use_tc_tiling_on_sc=False),
        out_type=[
            jax.ShapeDtypeStruct((B, 16), jnp.float32),
            jax.ShapeDtypeStruct((B, 16), jnp.float32),
        ],
        scratch_types=[
            pltpu.VMEM((nchunks, _CHUNK), jnp.int32),
            pltpu.VMEM((bpw,), jnp.int32),
            pltpu.VMEM((4 * nchunks, _CHUNK), jnp.int32),
            pltpu.VMEM((16, bpw), jnp.float32),
            pltpu.VMEM((8, 16), jnp.float32),
            pltpu.VMEM((8, 16), jnp.float32),
            pltpu.VMEM((bpw, 16), jnp.float32),
            pltpu.VMEM((bpw, 16), jnp.float32),
        ] + [pltpu.SemaphoreType.DMA] * (bpw // _CHUNK),
    )
    def k(*refs):
        (fidx_hbm, cidx_hbm) = refs[0:2]
        rtabs = refs[2:6]
        (cam_hbm, proj_hbm, pose_out, proj_out,
         fidx_v, cidx_v, idx_v, planes_v, cam_t, proj_t,
         pose_v, proj_v) = refs[6:18]
        sems = refs[18:]
        wid = lax.axis_index("s") * _NC + lax.axis_index("c")
        base = wid * bpw

        pltpu.sync_copy(fidx_hbm.at[pl.ds(wid * nchunks, nchunks)], fidx_v)

        lanes = lax.iota(jnp.int32, _L)

        copies = []
        for j in range(nchunks):
            def build(gg, carry, j=j):
                f = fidx_v[j, pl.ds(gg * _L, _L)]
                mb = (f >> 7) * (4 * _FT) + (f & (_FT - 1))
                tb = f + (4 * vm - vm)
                tail = f >= vm
                for c in range(4):
                    ic = jnp.where(tail, tb + c * (V - vm), mb + c * _FT)
                    idx_v[c * nchunks + j, pl.ds(gg * _L, _L)] = ic
                return carry
            lax.fori_loop(0, _CHUNK // _L, build, 0)
            copies.append([
                pltpu.async_copy(
                    rtabs[r].at[idx_v.at[c * nchunks + j]],
                    planes_v.at[r * 4 + c, pl.ds(j * _CHUNK, _CHUNK)],
                    sems[j],
                )
                for r in range(4)
                for c in range(4)
            ])

        pltpu.sync_copy(cidx_hbm.at[pl.ds(base, bpw)], cidx_v)
        pltpu.sync_copy(cam_hbm, cam_t)
        pltpu.sync_copy(proj_hbm, proj_t)

        def group(g, carry):
            item = g * _L + lanes
            cam_i = cidx_v[pl.ds(g * _L, _L)]
            for e in range(9):
                pe = plsc.load_gather(proj_t, [cam_i, _full(e)])
                plsc.store_scatter(proj_v, [item, _full(e)], pe)
            rig_e = [[planes_v[4 * kk + cc, pl.ds(g * _L, _L)]
                      for cc in range(4)] for kk in range(4)]
            for r in range(4):
                cam_k = [plsc.load_gather(cam_t, [cam_i, _full(4 * r + kk)])
                         for kk in range(4)]
                for cc in range(4):
                    acc = cam_k[0] * rig_e[0][cc]
                    acc = acc + cam_k[1] * rig_e[1][cc]
                    acc = acc + cam_k[2] * rig_e[2][cc]
                    acc = acc + cam_k[3] * rig_e[3][cc]
                    plsc.store_scatter(pose_v, [item, _full(4 * r + cc)], acc)
            return carry

        gpc = _CHUNK // _L
        for j in range(nchunks):
            for cp in copies[j]:
                cp.wait()
            lax.fori_loop(j * gpc, (j + 1) * gpc, group, 0)

        pltpu.sync_copy(pose_v, pose_out.at[pl.ds(base, bpw)])
        pltpu.sync_copy(proj_v, proj_out.at[pl.ds(base, bpw)])

    return k


def kernel(image_idx, rig_t_world, camera_t_rig, projection):
    B = image_idx.shape[0]
    V = rig_t_world.shape[0]
    vm = V // _FT * _FT
    fidx = image_idx[:, 0].astype(jnp.int32).reshape(B // _CHUNK, _CHUNK)
    cidx = image_idx[:, 1].astype(jnp.int32)
    rig_T = jnp.transpose(rig_t_world, (1, 2, 0))
    rtabs = [
        jnp.concatenate([
            jnp.transpose(rig_T[r, :, :vm].reshape(4, vm // _FT, _FT),
                          (1, 0, 2)).reshape(-1),
            rig_T[r, :, vm:].reshape(-1),
        ])
        for r in range(4)
    ]
    cam_flat = camera_t_rig.reshape(8, 16)
    proj_pad = jnp.pad(projection.reshape(8, 9), ((0, 0), (0, 7)))
    pose, proj = _make_sc_kernel(B, V)(fidx, cidx, *rtabs, cam_flat,
                                       proj_pad)
    return pose.reshape(B, 4, 4), proj[:, :9].reshape(B, 3, 3)


# speedup vs baseline: 1.3570x; 1.3570x over previous
import functools

import jax
import jax.numpy as jnp
from jax import lax
from jax.experimental import pallas as pl
from jax.experimental.pallas import tpu as pltpu
from jax.experimental.pallas import tpu_sc as plsc

_info = plsc.get_sparse_core_info()
_NC, _NS, _L = _info.num_cores, _info.num_subcores, _info.num_lanes
_NW = _NC * _NS
_CHUNK = 128


def _full(v):
    return jnp.full((_L,), v, jnp.int32)


@functools.cache
def _make_sc_kernel(B, V):
    bpw = B // _NW
    nchunks = bpw // _CHUNK
    mesh = plsc.VectorSubcoreMesh(core_axis_name="c", subcore_axis_name="s")

    @functools.partial(
        pl.kernel,
        mesh=mesh,
        compiler_params=pltpu.CompilerParams(
            needs_layout_passes=False, use_tc_tiling_on_sc=False),
        out_type=[
            jax.ShapeDtypeStruct((B, 16), jnp.float32),
            jax.ShapeDtypeStruct((B, 16), jnp.float32),
        ],
        scratch_types=[
            pltpu.VMEM((nchunks, _CHUNK), jnp.int32),
            pltpu.VMEM((bpw,), jnp.int32),
            pltpu.VMEM((16, bpw), jnp.float32),
            pltpu.VMEM((8, 16), jnp.float32),
            pltpu.VMEM((8, 16), jnp.float32),
            pltpu.VMEM((bpw, 16), jnp.float32),
            pltpu.VMEM((bpw, 16), jnp.float32),
        ] + [pltpu.SemaphoreType.DMA] * (bpw // _CHUNK),
    )
    def k(*refs):
        (fidx_hbm, cidx_hbm) = refs[0:2]
        rtabs = refs[2:6]
        tabs = [rtabs[e // 4].at[e % 4] for e in range(16)]
        (cam_hbm, proj_hbm, pose_out, proj_out,
         fidx_v, cidx_v, planes_v, cam_t, proj_t, pose_v, proj_v) = refs[6:17]
        sems = refs[17:]
        wid = lax.axis_index("s") * _NC + lax.axis_index("c")
        base = wid * bpw

        pltpu.sync_copy(fidx_hbm.at[pl.ds(wid * nchunks, nchunks)], fidx_v)

        copies = [
            [
                pltpu.async_copy(
                    tabs[e].at[fidx_v.at[j]],
                    planes_v.at[e, pl.ds(j * _CHUNK, _CHUNK)],
                    sems[j],
                )
                for e in range(16)
            ]
            for j in range(nchunks)
        ]

        pltpu.sync_copy(cidx_hbm.at[pl.ds(base, bpw)], cidx_v)
        pltpu.sync_copy(cam_hbm, cam_t)
        pltpu.sync_copy(proj_hbm, proj_t)

        lanes = lax.iota(jnp.int32, _L)

        def group(g, carry):
            item = g * _L + lanes
            cam_i = cidx_v[pl.ds(g * _L, _L)]
            for e in range(9):
                pe = plsc.load_gather(proj_t, [cam_i, _full(e)])
                plsc.store_scatter(proj_v, [item, _full(e)], pe)
            rig_e = [[planes_v[4 * kk + cc, pl.ds(g * _L, _L)]
                      for cc in range(4)] for kk in range(4)]
            for r in range(4):
                cam_k = [plsc.load_gather(cam_t, [cam_i, _full(4 * r + kk)])
                         for kk in range(4)]
                for cc in range(4):
                    acc = cam_k[0] * rig_e[0][cc]
                    acc = acc + cam_k[1] * rig_e[1][cc]
                    acc = acc + cam_k[2] * rig_e[2][cc]
                    acc = acc + cam_k[3] * rig_e[3][cc]
                    plsc.store_scatter(pose_v, [item, _full(4 * r + cc)], acc)
            return carry

        gpc = _CHUNK // _L
        for j in range(nchunks):
            for cp in copies[j]:
                cp.wait()
            lax.fori_loop(j * gpc, (j + 1) * gpc, group, 0)

        pltpu.sync_copy(pose_v, pose_out.at[pl.ds(base, bpw)])
        pltpu.sync_copy(proj_v, proj_out.at[pl.ds(base, bpw)])

    return k


def kernel(image_idx, rig_t_world, camera_t_rig, projection):
    B = image_idx.shape[0]
    V = rig_t_world.shape[0]
    fidx = image_idx[:, 0].astype(jnp.int32).reshape(B // _CHUNK, _CHUNK)
    cidx = image_idx[:, 1].astype(jnp.int32)
    rig_T = jnp.transpose(rig_t_world, (1, 2, 0))
    planes = [rig_T[r] for r in range(4)]
    cam_flat = camera_t_rig.reshape(8, 16)
    proj_pad = jnp.pad(projection.reshape(8, 9), ((0, 0), (0, 7)))
    pose, proj = _make_sc_kernel(B, V)(fidx, cidx, *planes, cam_flat,
                                       proj_pad)
    return pose.reshape(B, 4, 4), proj[:, :9].reshape(B, 3, 3)
